# baseline (speedup 1.0000x reference)
import functools

import jax
import jax.numpy as jnp
from jax import lax
from jax.experimental import pallas as pl
from jax.experimental.pallas import tpu as pltpu
from jax.experimental.pallas import tpu_sc as plsc

RW = 8
NG = 3
SUBW = 128
SUB = 8
SUPER = SUB * SUBW


def _sc_mesh():
    return plsc.VectorSubcoreMesh(core_axis_name="c", subcore_axis_name="s")


def _num_cores_subcores():
    try:
        info = plsc.get_sparse_core_info()
        return info.num_cores, info.num_subcores
    except Exception:
        return 2, 16


def _make_prop(n, nacc, nsup, nc, ns):
    zblks = nacc // (ns * SUBW)
    wb = nacc // ns
    nsup2 = nsup // nc
    npair = nsup2 // 4

    @functools.partial(
        pl.kernel,
        out_type=jax.ShapeDtypeStruct((nc, NG, nacc, RW), jnp.float32),
        mesh=_sc_mesh(),
        compiler_params=pltpu.CompilerParams(use_tc_tiling_on_sc=False),
        scratch_types=(
            [pltpu.VMEM((SUB, SUBW), jnp.int32) for _ in range(8)]
            + [pltpu.VMEM((SUB, SUBW, RW), jnp.float32) for _ in range(4)]
            + [pltpu.VMEM((SUBW, RW), jnp.float32),
               pltpu.VMEM_SHARED((nacc, RW), jnp.float32)]
            + [pltpu.SemaphoreType.DMA for _ in range(12)]
        ),
    )
    def prop(g_hbm, srcr_hbm, dstr_hbm, zero_hbm, out_hbm, *scr):
        srcb = scr[0:4]
        dstb = scr[4:8]
        rows = scr[8:12]
        zero_v = scr[12]
        acc_sh = scr[13]
        isem = scr[14:18]
        gsem = scr[18:22]
        ssem = scr[22:26]
        c = lax.axis_index("c")
        s = lax.axis_index("s")
        pltpu.sync_copy(zero_hbm, zero_v)
        base = (s * nsup + c * nsup2) * SUB

        def fire_idx(gq, i, b):
            r0 = base + i * SUB
            pltpu.async_copy(srcr_hbm.at[gq, pl.ds(r0, SUB)], srcb[b],
                             isem[b])
            pltpu.async_copy(dstr_hbm.at[pl.ds(r0, SUB)], dstb[b], isem[b])

        def drain_idx(gq, i, b):
            r0 = base + i * SUB
            pltpu.make_async_copy(srcr_hbm.at[gq, pl.ds(r0, SUB)], srcb[b],
                                  isem[b]).wait()
            pltpu.make_async_copy(dstr_hbm.at[pl.ds(r0, SUB)], dstb[b],
                                  isem[b]).wait()

        def fire_gather(b):
            for j in range(SUB):
                pltpu.async_copy(g_hbm.at[srcb[b].at[j]], rows[b].at[j],
                                 gsem[b])

        def drain_gather(b):
            for j in range(SUB):
                pltpu.make_async_copy(g_hbm.at[srcb[b].at[j]],
                                      rows[b].at[j], gsem[b]).wait()

        def fire_scat(b):
            for j in range(SUB):
                pltpu.async_copy(rows[b].at[j], acc_sh.at[dstb[b].at[j]],
                                 ssem[b], add=True)

        def drain_scat(b):
            for j in range(SUB):
                pltpu.make_async_copy(rows[b].at[j],
                                      acc_sh.at[dstb[b].at[j]],
                                      ssem[b]).wait()

        def gbody(gq, carry0):
            def zbody(r, carry):
                pltpu.sync_copy(
                    zero_v, acc_sh.at[pl.ds((s * zblks + r) * SUBW, SUBW)])
                return carry

            lax.fori_loop(0, zblks, zbody, 0)
            plsc.subcore_barrier()

            fire_idx(gq, 0, 0)
            fire_idx(gq, 1, 1)
            drain_idx(gq, 0, 0)
            fire_gather(0)

            def body(p, carry):
                i0 = p * 4
                for q in range(4):
                    i = i0 + q
                    b = q
                    b1 = (q + 1) % 4
                    b2 = (q + 2) % 4
                    if q >= 2:
                        drain_scat(b2)

                        @pl.when(p < npair - 1)
                        def _():
                            fire_idx(gq, i + 2, b2)
                    else:
                        @pl.when(p > 0)
                        def _():
                            drain_scat(b2)

                        fire_idx(gq, i + 2, b2)
                    drain_gather(b)
                    fire_scat(b)
                    if q == 3:
                        @pl.when(p < npair - 1)
                        def _():
                            drain_idx(gq, i + 1, b1)
                            fire_gather(b1)
                    else:
                        drain_idx(gq, i + 1, b1)
                        fire_gather(b1)
                return carry

            lax.fori_loop(0, npair, body, 0)
            drain_scat(2)
            drain_scat(3)
            plsc.subcore_barrier()
            pltpu.sync_copy(acc_sh.at[pl.ds(s * wb, wb)],
                            out_hbm.at[c, gq, pl.ds(s * wb, wb)])
            plsc.subcore_barrier()
            return carry0

        lax.fori_loop(0, NG, gbody, 0)

    return prop


def _make_deg(n, nacc, nsupd, nc, ns):
    zblks = nacc // (ns * SUBW)
    wb = nacc // ns

    @functools.partial(
        pl.kernel,
        out_type=jax.ShapeDtypeStruct((nc, nacc, RW), jnp.float32),
        mesh=_sc_mesh(),
        compiler_params=pltpu.CompilerParams(use_tc_tiling_on_sc=False),
        scratch_types=[
            pltpu.VMEM((SUB, SUBW), jnp.int32),
            pltpu.VMEM((SUBW, RW), jnp.float32),
            pltpu.VMEM((SUBW, RW), jnp.float32),
            pltpu.VMEM_SHARED((nacc, RW), jnp.float32),
        ],
    )
    def deg(srcr_hbm, ones_hbm, zero_hbm, out_hbm,
            idx_v, ones_v, zero_v, acc_sh):
        c = lax.axis_index("c")
        s = lax.axis_index("s")

        pltpu.sync_copy(zero_hbm, zero_v)
        pltpu.sync_copy(ones_hbm, ones_v)

        def zbody(r, carry):
            pltpu.sync_copy(zero_v,
                            acc_sh.at[pl.ds((s * zblks + r) * SUBW, SUBW)])
            return carry

        lax.fori_loop(0, zblks, zbody, 0)
        plsc.subcore_barrier()

        def body(i, carry):
            row0 = ((c * ns + s) * nsupd + i) * SUB
            pltpu.sync_copy(srcr_hbm.at[pl.ds(row0, SUB)], idx_v)
            for j in range(SUB):
                pltpu.sync_copy(ones_v, acc_sh.at[idx_v.at[j]], add=True)
            return carry

        lax.fori_loop(0, nsupd, body, 0)
        plsc.subcore_barrier()
        pltpu.sync_copy(acc_sh.at[pl.ds(s * wb, wb)],
                        out_hbm.at[c, pl.ds(s * wb, wb)])

    return deg


def _prep0_body(x_ref, dp_ref, w0_ref, dinv_ref, g_ref, t0_ref, out_ref):
    deg = dp_ref[0, :, 0:1] + dp_ref[1, :, 0:1]
    dinv = jnp.where(deg > 0.0,
                     lax.rsqrt(jnp.maximum(deg, 1e-12)),
                     jnp.zeros_like(deg))
    dinv_ref[...] = dinv
    xb = x_ref[...]
    for g in range(NG):
        hg = xb[:, g * RW:(g + 1) * RW]
        t0_ref[g] = hg
        g_ref[g] = dinv * hg
    out_ref[...] = jnp.dot(xb, w0_ref[...],
                           preferred_element_type=jnp.float32)


def _step_body(first, with_g, *refs):
    if first:
        acc_ref, dinv_ref, wp_ref, outin_ref = refs[:4]
        orefs = refs[4:]
        tp2_ref = None
        scale = 1.0
    else:
        acc_ref, dinv_ref, tp2_ref, wp_ref, outin_ref = refs[:5]
        orefs = refs[5:]
        scale = 2.0
    if with_g:
        tk_ref, g_ref, outo_ref = orefs
    else:
        tk_ref, outo_ref = orefs
    dinv = dinv_ref[...]
    wp = wp_ref[...]
    o = outin_ref[...]
    for g in range(NG):
        acc_g = acc_ref[0, g] + acc_ref[1, g]
        t_g = (-scale) * dinv * acc_g
        if not first:
            t_g = t_g - tp2_ref[g]
        tk_ref[g] = t_g
        if with_g:
            g_ref[g] = dinv * t_g
        o = o + jnp.dot(t_g, wp[g * RW:(g + 1) * RW],
                        preferred_element_type=jnp.float32)
    outo_ref[...] = o


def _final_body(h_ref, b1_ref, gam_ref, bet_ref, mu_ref, var_ref,
                wm_ref, bm_ref, y_ref):
    o = h_ref[...] + b1_ref[...]
    o = (o - mu_ref[...]) * lax.rsqrt(var_ref[...] + 1e-5) * gam_ref[...] \
        + bet_ref[...]
    h = jnp.maximum(o, 0.0)
    y_ref[...] = jnp.dot(h, wm_ref[...],
                         preferred_element_type=jnp.float32) + bm_ref[...]


def kernel(x, edge_index, W1, b1, gamma, beta, bn_mean, bn_var, Wmix, bmix):
    n, n_in = x.shape
    e = edge_index.shape[1]
    kblk = W1.shape[0]
    n_hid = W1.shape[2]
    n_out = Wmix.shape[2]
    nc, ns = _num_cores_subcores()

    per_tile = ns * SUPER
    nsup = -(-e // per_tile)
    nsup = -(-nsup // (4 * nc)) * (4 * nc)
    nsupd = nsup // nc
    epad = ns * nsup * SUPER
    pad = epad - e

    src = edge_index[0]
    dst = edge_index[1]
    src_g = jnp.concatenate([src, jnp.zeros((pad,), jnp.int32)])
    src_n = jnp.concatenate([src, jnp.full((pad,), n, jnp.int32)])
    dst_n = jnp.concatenate([dst, jnp.full((pad,), n, jnp.int32)])
    src3r = jnp.stack([src_g + g * n for g in range(NG)]) \
               .reshape(NG, epad // SUBW, SUBW)
    dstr = dst_n.reshape(epad // SUBW, SUBW)
    srcdr = src_n.reshape(epad // SUBW, SUBW)

    nacc = -(-(n + 1) // (ns * SUBW)) * (ns * SUBW)

    zero128 = jnp.zeros((SUBW, RW), jnp.float32)
    ones128 = jnp.ones((SUBW, RW), jnp.float32)

    deg_fn = _make_deg(n, nacc, nsupd, nc, ns)
    prop_fn = _make_prop(n, nacc, nsup, nc, ns)

    degp = deg_fn(srcdr, ones128, zero128)

    bsz = 2000
    grid = (n // bsz,)
    f32 = jnp.float32

    spec_pack = pl.BlockSpec((NG, bsz, RW), lambda b: (0, b, 0))
    spec_acc = pl.BlockSpec((nc, NG, bsz, RW), lambda b: (0, 0, b, 0))
    spec_deg = pl.BlockSpec((nc, bsz, RW), lambda b: (0, b, 0))
    spec_x = pl.BlockSpec((bsz, n_in), lambda b: (b, 0))
    spec_dinv = pl.BlockSpec((bsz, 1), lambda b: (b, 0))
    spec_out = pl.BlockSpec((bsz, n_hid), lambda b: (b, 0))
    spec_w = pl.BlockSpec((n_in, n_hid), lambda b: (0, 0))

    dinv, g, t_prev2, out = pl.pallas_call(
        _prep0_body,
        grid=grid,
        in_specs=[spec_x, spec_deg, spec_w],
        out_specs=[spec_dinv, spec_pack, spec_pack, spec_out],
        out_shape=[
            jax.ShapeDtypeStruct((n, 1), f32),
            jax.ShapeDtypeStruct((NG, n, RW), f32),
            jax.ShapeDtypeStruct((NG, n, RW), f32),
            jax.ShapeDtypeStruct((n, n_hid), f32),
        ],
    )(x, degp, W1[0])

    t_prev1 = None
    for k in range(1, kblk):
        acc = prop_fn(g.reshape(NG * n, RW), src3r, dstr, zero128)
        first = (k == 1)
        with_g = (k < kblk - 1)
        out_shapes = [jax.ShapeDtypeStruct((NG, n, RW), f32)]
        out_specs = [spec_pack]
        if with_g:
            out_shapes.append(jax.ShapeDtypeStruct((NG, n, RW), f32))
            out_specs.append(spec_pack)
        out_shapes.append(jax.ShapeDtypeStruct((n, n_hid), f32))
        out_specs.append(spec_out)
        if first:
            in_specs = [spec_acc, spec_dinv, spec_w, spec_out]
            operands = (acc, dinv, W1[k], out)
            alias = {3: len(out_shapes) - 1}
        else:
            in_specs = [spec_acc, spec_dinv, spec_pack, spec_w, spec_out]
            operands = (acc, dinv, t_prev2, W1[k], out)
            alias = {4: len(out_shapes) - 1}
        res = pl.pallas_call(
            functools.partial(_step_body, first, with_g),
            grid=grid,
            in_specs=in_specs,
            out_specs=out_specs,
            out_shape=out_shapes,
            input_output_aliases=alias,
        )(*operands)
        if with_g:
            t_k, g, out = res
        else:
            t_k, out = res
        if first:
            t_prev1 = t_k
        else:
            t_prev2, t_prev1 = t_prev1, t_k

    spec_vec = pl.BlockSpec((1, n_hid), lambda b: (0, 0))
    spec_wm = pl.BlockSpec((n_hid, n_out), lambda b: (0, 0))
    spec_bm = pl.BlockSpec((1, n_out), lambda b: (0, 0))
    spec_y = pl.BlockSpec((bsz, n_out), lambda b: (b, 0))

    y = pl.pallas_call(
        _final_body,
        grid=grid,
        in_specs=[spec_out, spec_vec, spec_vec, spec_vec, spec_vec,
                  spec_vec, spec_wm, spec_bm],
        out_specs=spec_y,
        out_shape=jax.ShapeDtypeStruct((n, n_out), f32),
    )(out, b1.reshape(1, n_hid), gamma.reshape(1, n_hid),
      beta.reshape(1, n_hid), bn_mean.reshape(1, n_hid),
      bn_var.reshape(1, n_hid), Wmix[0], bmix.reshape(1, n_out))
    return y


# speedup vs baseline: 1.0478x; 1.0478x over previous
import functools

import jax
import jax.numpy as jnp
from jax import lax
from jax.experimental import pallas as pl
from jax.experimental.pallas import tpu as pltpu
from jax.experimental.pallas import tpu_sc as plsc

RW = 8
NG = 3
SUBW = 128
SUB = 8
SUPER = SUB * SUBW


def _sc_mesh():
    return plsc.VectorSubcoreMesh(core_axis_name="c", subcore_axis_name="s")


def _num_cores_subcores():
    try:
        info = plsc.get_sparse_core_info()
        return info.num_cores, info.num_subcores
    except Exception:
        return 2, 16


def _make_prop(n, nacc, nsup, nc, ns, frac0=0.7):
    zblks = nacc // (ns * SUBW)
    wb = nacc // ns
    nsup0 = int(round(nsup * frac0 / 4.0)) * 4
    nsup1 = nsup - nsup0

    @functools.partial(
        pl.kernel,
        out_type=jax.ShapeDtypeStruct((nc, NG, nacc, RW), jnp.float32),
        mesh=_sc_mesh(),
        compiler_params=pltpu.CompilerParams(use_tc_tiling_on_sc=False),
        scratch_types=(
            [pltpu.VMEM((SUB, SUBW), jnp.int32) for _ in range(8)]
            + [pltpu.VMEM((SUB, SUBW, RW), jnp.float32) for _ in range(4)]
            + [pltpu.VMEM((SUBW, RW), jnp.float32),
               pltpu.VMEM_SHARED((nacc, RW), jnp.float32)]
            + [pltpu.SemaphoreType.DMA for _ in range(12)]
        ),
    )
    def prop(g_hbm, srcr_hbm, dstr_hbm, zero_hbm, out_hbm, *scr):
        srcb = scr[0:4]
        dstb = scr[4:8]
        rows = scr[8:12]
        zero_v = scr[12]
        acc_sh = scr[13]
        isem = scr[14:18]
        gsem = scr[18:22]
        ssem = scr[22:26]
        c = lax.axis_index("c")
        s = lax.axis_index("s")
        pltpu.sync_copy(zero_hbm, zero_v)
        base = (s * nsup + jnp.where(c == 0, 0, nsup0)) * SUB
        npair = jnp.where(c == 0, nsup0 // 4, nsup1 // 4)

        def fire_idx(gq, i, b):
            r0 = base + i * SUB
            pltpu.async_copy(srcr_hbm.at[gq, pl.ds(r0, SUB)], srcb[b],
                             isem[b])
            pltpu.async_copy(dstr_hbm.at[pl.ds(r0, SUB)], dstb[b], isem[b])

        def drain_idx(gq, i, b):
            r0 = base + i * SUB
            pltpu.make_async_copy(srcr_hbm.at[gq, pl.ds(r0, SUB)], srcb[b],
                                  isem[b]).wait()
            pltpu.make_async_copy(dstr_hbm.at[pl.ds(r0, SUB)], dstb[b],
                                  isem[b]).wait()

        def fire_gather(b):
            for j in range(SUB):
                pltpu.async_copy(g_hbm.at[srcb[b].at[j]], rows[b].at[j],
                                 gsem[b])

        def drain_gather(b):
            for j in range(SUB):
                pltpu.make_async_copy(g_hbm.at[srcb[b].at[j]],
                                      rows[b].at[j], gsem[b]).wait()

        def fire_scat(b):
            for j in range(SUB):
                pltpu.async_copy(rows[b].at[j], acc_sh.at[dstb[b].at[j]],
                                 ssem[b], add=True)

        def drain_scat(b):
            for j in range(SUB):
                pltpu.make_async_copy(rows[b].at[j],
                                      acc_sh.at[dstb[b].at[j]],
                                      ssem[b]).wait()

        def gbody(gq, carry0):
            def zbody(r, carry):
                pltpu.sync_copy(
                    zero_v, acc_sh.at[pl.ds((s * zblks + r) * SUBW, SUBW)])
                return carry

            lax.fori_loop(0, zblks, zbody, 0)
            plsc.subcore_barrier()

            fire_idx(gq, 0, 0)
            fire_idx(gq, 1, 1)
            drain_idx(gq, 0, 0)
            fire_gather(0)

            def body(p, carry):
                i0 = p * 4
                for q in range(4):
                    i = i0 + q
                    b = q
                    b1 = (q + 1) % 4
                    b2 = (q + 2) % 4
                    if q >= 2:
                        drain_scat(b2)

                        @pl.when(p < npair - 1)
                        def _():
                            fire_idx(gq, i + 2, b2)
                    else:
                        @pl.when(p > 0)
                        def _():
                            drain_scat(b2)

                        fire_idx(gq, i + 2, b2)
                    drain_gather(b)
                    fire_scat(b)
                    if q == 3:
                        @pl.when(p < npair - 1)
                        def _():
                            drain_idx(gq, i + 1, b1)
                            fire_gather(b1)
                    else:
                        drain_idx(gq, i + 1, b1)
                        fire_gather(b1)
                return carry

            lax.fori_loop(0, npair, body, 0)
            drain_scat(2)
            drain_scat(3)
            plsc.subcore_barrier()
            pltpu.sync_copy(acc_sh.at[pl.ds(s * wb, wb)],
                            out_hbm.at[c, gq, pl.ds(s * wb, wb)])
            plsc.subcore_barrier()
            return carry0

        lax.fori_loop(0, NG, gbody, 0)

    return prop


def _make_deg(n, nacc, nsupd, nc, ns):
    zblks = nacc // (ns * SUBW)
    wb = nacc // ns

    @functools.partial(
        pl.kernel,
        out_type=jax.ShapeDtypeStruct((nc, nacc, RW), jnp.float32),
        mesh=_sc_mesh(),
        compiler_params=pltpu.CompilerParams(use_tc_tiling_on_sc=False),
        scratch_types=[
            pltpu.VMEM((SUB, SUBW), jnp.int32),
            pltpu.VMEM((SUBW, RW), jnp.float32),
            pltpu.VMEM((SUBW, RW), jnp.float32),
            pltpu.VMEM_SHARED((nacc, RW), jnp.float32),
        ],
    )
    def deg(srcr_hbm, ones_hbm, zero_hbm, out_hbm,
            idx_v, ones_v, zero_v, acc_sh):
        c = lax.axis_index("c")
        s = lax.axis_index("s")

        pltpu.sync_copy(zero_hbm, zero_v)
        pltpu.sync_copy(ones_hbm, ones_v)

        def zbody(r, carry):
            pltpu.sync_copy(zero_v,
                            acc_sh.at[pl.ds((s * zblks + r) * SUBW, SUBW)])
            return carry

        lax.fori_loop(0, zblks, zbody, 0)
        plsc.subcore_barrier()

        def body(i, carry):
            row0 = ((c * ns + s) * nsupd + i) * SUB
            pltpu.sync_copy(srcr_hbm.at[pl.ds(row0, SUB)], idx_v)
            for j in range(SUB):
                pltpu.sync_copy(ones_v, acc_sh.at[idx_v.at[j]], add=True)
            return carry

        lax.fori_loop(0, nsupd, body, 0)
        plsc.subcore_barrier()
        pltpu.sync_copy(acc_sh.at[pl.ds(s * wb, wb)],
                        out_hbm.at[c, pl.ds(s * wb, wb)])

    return deg


def _prep0_body(x_ref, dp_ref, w0_ref, dinv_ref, g_ref, t0_ref, out_ref):
    deg = dp_ref[0, :, 0:1] + dp_ref[1, :, 0:1]
    dinv = jnp.where(deg > 0.0,
                     lax.rsqrt(jnp.maximum(deg, 1e-12)),
                     jnp.zeros_like(deg))
    dinv_ref[...] = dinv
    xb = x_ref[...]
    for g in range(NG):
        hg = xb[:, g * RW:(g + 1) * RW]
        t0_ref[g] = hg
        g_ref[g] = dinv * hg
    out_ref[...] = jnp.dot(xb, w0_ref[...],
                           preferred_element_type=jnp.float32)


def _step_body(first, with_g, *refs):
    if first:
        acc_ref, dinv_ref, wp_ref, outin_ref = refs[:4]
        orefs = refs[4:]
        tp2_ref = None
        scale = 1.0
    else:
        acc_ref, dinv_ref, tp2_ref, wp_ref, outin_ref = refs[:5]
        orefs = refs[5:]
        scale = 2.0
    if with_g:
        tk_ref, g_ref, outo_ref = orefs
    else:
        tk_ref, outo_ref = orefs
    dinv = dinv_ref[...]
    wp = wp_ref[...]
    o = outin_ref[...]
    for g in range(NG):
        acc_g = acc_ref[0, g] + acc_ref[1, g]
        t_g = (-scale) * dinv * acc_g
        if not first:
            t_g = t_g - tp2_ref[g]
        tk_ref[g] = t_g
        if with_g:
            g_ref[g] = dinv * t_g
        o = o + jnp.dot(t_g, wp[g * RW:(g + 1) * RW],
                        preferred_element_type=jnp.float32)
    outo_ref[...] = o


def _final_body(h_ref, b1_ref, gam_ref, bet_ref, mu_ref, var_ref,
                wm_ref, bm_ref, y_ref):
    o = h_ref[...] + b1_ref[...]
    o = (o - mu_ref[...]) * lax.rsqrt(var_ref[...] + 1e-5) * gam_ref[...] \
        + bet_ref[...]
    h = jnp.maximum(o, 0.0)
    y_ref[...] = jnp.dot(h, wm_ref[...],
                         preferred_element_type=jnp.float32) + bm_ref[...]


def kernel(x, edge_index, W1, b1, gamma, beta, bn_mean, bn_var, Wmix, bmix):
    n, n_in = x.shape
    e = edge_index.shape[1]
    kblk = W1.shape[0]
    n_hid = W1.shape[2]
    n_out = Wmix.shape[2]
    nc, ns = _num_cores_subcores()

    per_tile = ns * SUPER
    nsup = -(-e // per_tile)
    nsup = -(-nsup // (4 * nc)) * (4 * nc)
    nsupd = nsup // nc
    epad = ns * nsup * SUPER
    pad = epad - e

    src = edge_index[0]
    dst = edge_index[1]
    src_g = jnp.concatenate([src, jnp.zeros((pad,), jnp.int32)])
    src_n = jnp.concatenate([src, jnp.full((pad,), n, jnp.int32)])
    dst_n = jnp.concatenate([dst, jnp.full((pad,), n, jnp.int32)])
    src3r = jnp.stack([src_g + g * n for g in range(NG)]) \
               .reshape(NG, epad // SUBW, SUBW)
    dstr = dst_n.reshape(epad // SUBW, SUBW)
    srcdr = src_n.reshape(epad // SUBW, SUBW)

    nacc = -(-(n + 1) // (ns * SUBW)) * (ns * SUBW)

    zero128 = jnp.zeros((SUBW, RW), jnp.float32)
    ones128 = jnp.ones((SUBW, RW), jnp.float32)

    deg_fn = _make_deg(n, nacc, nsupd, nc, ns)
    prop_fn = _make_prop(n, nacc, nsup, nc, ns)

    degp = deg_fn(srcdr, ones128, zero128)

    bsz = 2000
    grid = (n // bsz,)
    f32 = jnp.float32

    spec_pack = pl.BlockSpec((NG, bsz, RW), lambda b: (0, b, 0))
    spec_acc = pl.BlockSpec((nc, NG, bsz, RW), lambda b: (0, 0, b, 0))
    spec_deg = pl.BlockSpec((nc, bsz, RW), lambda b: (0, b, 0))
    spec_x = pl.BlockSpec((bsz, n_in), lambda b: (b, 0))
    spec_dinv = pl.BlockSpec((bsz, 1), lambda b: (b, 0))
    spec_out = pl.BlockSpec((bsz, n_hid), lambda b: (b, 0))
    spec_w = pl.BlockSpec((n_in, n_hid), lambda b: (0, 0))

    dinv, g, t_prev2, out = pl.pallas_call(
        _prep0_body,
        grid=grid,
        in_specs=[spec_x, spec_deg, spec_w],
        out_specs=[spec_dinv, spec_pack, spec_pack, spec_out],
        out_shape=[
            jax.ShapeDtypeStruct((n, 1), f32),
            jax.ShapeDtypeStruct((NG, n, RW), f32),
            jax.ShapeDtypeStruct((NG, n, RW), f32),
            jax.ShapeDtypeStruct((n, n_hid), f32),
        ],
    )(x, degp, W1[0])

    t_prev1 = None
    for k in range(1, kblk):
        acc = prop_fn(g.reshape(NG * n, RW), src3r, dstr, zero128)
        first = (k == 1)
        with_g = (k < kblk - 1)
        out_shapes = [jax.ShapeDtypeStruct((NG, n, RW), f32)]
        out_specs = [spec_pack]
        if with_g:
            out_shapes.append(jax.ShapeDtypeStruct((NG, n, RW), f32))
            out_specs.append(spec_pack)
        out_shapes.append(jax.ShapeDtypeStruct((n, n_hid), f32))
        out_specs.append(spec_out)
        if first:
            in_specs = [spec_acc, spec_dinv, spec_w, spec_out]
            operands = (acc, dinv, W1[k], out)
            alias = {3: len(out_shapes) - 1}
        else:
            in_specs = [spec_acc, spec_dinv, spec_pack, spec_w, spec_out]
            operands = (acc, dinv, t_prev2, W1[k], out)
            alias = {4: len(out_shapes) - 1}
        res = pl.pallas_call(
            functools.partial(_step_body, first, with_g),
            grid=grid,
            in_specs=in_specs,
            out_specs=out_specs,
            out_shape=out_shapes,
            input_output_aliases=alias,
        )(*operands)
        if with_g:
            t_k, g, out = res
        else:
            t_k, out = res
        if first:
            t_prev1 = t_k
        else:
            t_prev2, t_prev1 = t_prev1, t_k

    spec_vec = pl.BlockSpec((1, n_hid), lambda b: (0, 0))
    spec_wm = pl.BlockSpec((n_hid, n_out), lambda b: (0, 0))
    spec_bm = pl.BlockSpec((1, n_out), lambda b: (0, 0))
    spec_y = pl.BlockSpec((bsz, n_out), lambda b: (b, 0))

    y = pl.pallas_call(
        _final_body,
        grid=grid,
        in_specs=[spec_out, spec_vec, spec_vec, spec_vec, spec_vec,
                  spec_vec, spec_wm, spec_bm],
        out_specs=spec_y,
        out_shape=jax.ShapeDtypeStruct((n, n_out), f32),
    )(out, b1.reshape(1, n_hid), gamma.reshape(1, n_hid),
      beta.reshape(1, n_hid), bn_mean.reshape(1, n_hid),
      bn_var.reshape(1, n_hid), Wmix[0], bmix.reshape(1, n_out))
    return y


# speedup vs baseline: 1.8990x; 1.8123x over previous
import functools

import jax
import jax.numpy as jnp
from jax import lax
from jax.experimental import pallas as pl
from jax.experimental.pallas import tpu as pltpu
from jax.experimental.pallas import tpu_sc as plsc

RW = 16
FW = 12
NG = 2
SUBW = 128
SUB = 4
SUPER = SUB * SUBW


def _sc_mesh():
    return plsc.VectorSubcoreMesh(core_axis_name="c", subcore_axis_name="s")


def _num_cores_subcores():
    try:
        info = plsc.get_sparse_core_info()
        return info.num_cores, info.num_subcores
    except Exception:
        return 2, 16


def _make_mega(n, nacc, nsup, nc, ns, kblk):
    zblks = nacc // (ns * 64)
    wb = nacc // ns
    ublks = wb // SUBW

    @functools.partial(
        pl.kernel,
        out_type=[
            jax.ShapeDtypeStruct((kblk - 1, NG, nacc, RW), jnp.float32),
            jax.ShapeDtypeStruct((2 * NG * nacc, RW), jnp.float32),
        ],
        mesh=_sc_mesh(),
        compiler_params=pltpu.CompilerParams(
            use_tc_tiling_on_sc=False,
            internal_scratch_in_bytes=64 * 1024),
        scratch_types=(
            [pltpu.VMEM((SUB, SUBW), jnp.int32) for _ in range(6)]
            + [pltpu.VMEM((SUB, SUBW, RW), jnp.float32) for _ in range(2)]
            + [pltpu.VMEM((64, RW), jnp.float32)]
            + [pltpu.VMEM((SUBW, RW), jnp.float32) for _ in range(3)]
            + [pltpu.VMEM_SHARED((nacc, RW), jnp.float32)]
            + [pltpu.SemaphoreType.DMA for _ in range(7)]
        ),
    )
    def mega(g0t_hbm, srcr_hbm, dstr_hbm, d2r_hbm, zero_hbm,
             accs_hbm, gbuf_hbm, *scr):
        srcb = scr[0:3]
        dstb = scr[3:6]
        rows = scr[6:8]
        zero_v = scr[8]
        ub_a = scr[9]
        ub_b = scr[10]
        ub_d = scr[11]
        acc_sh = scr[12]
        isem = scr[13:16]
        gsem = scr[16:18]
        ssem = scr[18:20]
        c = lax.axis_index("c")
        s = lax.axis_index("s")
        gq = c
        pltpu.sync_copy(zero_hbm, zero_v)
        base = s * nsup * SUB
        r0w = s * wb

        def cbody(blk, carry):
            r0 = r0w + blk * SUBW
            pltpu.sync_copy(g0t_hbm.at[gq, pl.ds(r0, SUBW)], ub_a)
            pltpu.sync_copy(ub_a, gbuf_hbm.at[pl.ds(gq * nacc + r0, SUBW)])
            return carry

        lax.fori_loop(0, ublks, cbody, 0)
        plsc.subcore_barrier()

        def fire_idx(srcsel, i, b):
            r0 = base + i * SUB
            pltpu.async_copy(srcr_hbm.at[srcsel, pl.ds(r0, SUB)], srcb[b],
                             isem[b])
            pltpu.async_copy(dstr_hbm.at[pl.ds(r0, SUB)], dstb[b], isem[b])

        def drain_idx(srcsel, i, b):
            r0 = base + i * SUB
            pltpu.make_async_copy(srcr_hbm.at[srcsel, pl.ds(r0, SUB)],
                                  srcb[b], isem[b]).wait()
            pltpu.make_async_copy(dstr_hbm.at[pl.ds(r0, SUB)], dstb[b],
                                  isem[b]).wait()

        def fire_gather(bi, br):
            for j in range(SUB):
                pltpu.async_copy(gbuf_hbm.at[srcb[bi].at[j]],
                                 rows[br].at[j], gsem[br])

        def drain_gather(bi, br):
            for j in range(SUB):
                pltpu.make_async_copy(gbuf_hbm.at[srcb[bi].at[j]],
                                      rows[br].at[j], gsem[br]).wait()

        def fire_scat(bi, br):
            for j in range(SUB):
                pltpu.async_copy(rows[br].at[j], acc_sh.at[dstb[bi].at[j]],
                                 ssem[br], add=True)

        def drain_scat(bi, br):
            for j in range(SUB):
                pltpu.make_async_copy(rows[br].at[j],
                                      acc_sh.at[dstb[bi].at[j]],
                                      ssem[br]).wait()

        def kbody(k, carry):
            cur = lax.rem(k - 1, 2)
            nxt = lax.rem(k, 2)
            srcsel = cur * NG + gq
            scl = jnp.where(k == 1, jnp.float32(1.0), jnp.float32(2.0))

            def zbody(r, carry2):
                pltpu.sync_copy(
                    zero_v, acc_sh.at[pl.ds((s * zblks + r) * 64, 64)])
                return carry2

            lax.fori_loop(0, zblks, zbody, 0)
            plsc.subcore_barrier()

            fire_idx(srcsel, 0, 0)
            fire_idx(srcsel, 1, 1)
            drain_idx(srcsel, 0, 0)
            fire_gather(0, 0)

            def body(p, carry2):
                i0 = p * 6
                for q in range(6):
                    i = i0 + q
                    br = q % 2
                    brn = (q + 1) % 2
                    bi = q % 3
                    bi1 = (q + 1) % 3
                    bi2 = (q + 2) % 3
                    drain_gather(bi, br)
                    fire_scat(bi, br)
                    if q == 0:
                        @pl.when(p > 0)
                        def _():
                            drain_scat(bi2, brn)
                    else:
                        drain_scat(bi2, brn)

                    @pl.when(i + 1 < nsup)
                    def _():
                        drain_idx(srcsel, i + 1, bi1)
                        fire_gather(bi1, brn)

                    @pl.when(i + 2 < nsup)
                    def _():
                        fire_idx(srcsel, i + 2, bi2)
                return carry2

            lax.fori_loop(0, nsup // 6, body, 0)
            drain_scat((nsup - 1) % 3, (nsup - 1) % 2)
            plsc.subcore_barrier()

            def ubody(blk, carry2):
                r0 = r0w + blk * SUBW
                pltpu.sync_copy(acc_sh.at[pl.ds(r0, SUBW)], ub_a)
                pltpu.sync_copy(
                    gbuf_hbm.at[pl.ds((nxt * NG + gq) * nacc + r0, SUBW)],
                    ub_b)
                pltpu.sync_copy(d2r_hbm.at[pl.ds(r0, SUBW)], ub_d)

                def vbody(v, carry3):
                    a = ub_a[v]
                    gp = ub_b[v]
                    d2 = ub_d[v]
                    gn = (-scl) * d2 * a - \
                        jnp.where(k > 1, gp, jnp.zeros_like(gp))
                    ub_b[v] = gn
                    return carry3

                lax.fori_loop(0, SUBW, vbody, 0)
                pltpu.sync_copy(
                    ub_b,
                    gbuf_hbm.at[pl.ds((nxt * NG + gq) * nacc + r0, SUBW)])
                pltpu.sync_copy(ub_a,
                                accs_hbm.at[k - 1, gq, pl.ds(r0, SUBW)])
                return carry2

            lax.fori_loop(0, ublks, ubody, 0)
            plsc.subcore_barrier()
            return carry

        lax.fori_loop(1, kblk, kbody, 0)

    return mega


def _make_deg(n, nacc, nsupd, nc, ns):
    zblks = nacc // (ns * 64)
    wb = nacc // ns

    @functools.partial(
        pl.kernel,
        out_type=jax.ShapeDtypeStruct((nc, nacc, RW), jnp.float32),
        mesh=_sc_mesh(),
        compiler_params=pltpu.CompilerParams(
            use_tc_tiling_on_sc=False,
            internal_scratch_in_bytes=64 * 1024),
        scratch_types=[
            pltpu.VMEM((SUB, SUBW), jnp.int32),
            pltpu.VMEM((SUBW, RW), jnp.float32),
            pltpu.VMEM((64, RW), jnp.float32),
            pltpu.VMEM_SHARED((nacc, RW), jnp.float32),
        ],
    )
    def deg(srcr_hbm, ones_hbm, zero_hbm, out_hbm,
            idx_v, ones_v, zero_v, acc_sh):
        c = lax.axis_index("c")
        s = lax.axis_index("s")

        pltpu.sync_copy(zero_hbm, zero_v)
        pltpu.sync_copy(ones_hbm, ones_v)

        def zbody(r, carry):
            pltpu.sync_copy(zero_v,
                            acc_sh.at[pl.ds((s * zblks + r) * 64, 64)])
            return carry

        lax.fori_loop(0, zblks, zbody, 0)
        plsc.subcore_barrier()

        def body(i, carry):
            row0 = ((c * ns + s) * nsupd + i) * SUB
            pltpu.sync_copy(srcr_hbm.at[pl.ds(row0, SUB)], idx_v)
            for j in range(SUB):
                pltpu.sync_copy(ones_v, acc_sh.at[idx_v.at[j]], add=True)
            return carry

        lax.fori_loop(0, nsupd, body, 0)
        plsc.subcore_barrier()
        pltpu.sync_copy(acc_sh.at[pl.ds(s * wb, wb)],
                        out_hbm.at[c, pl.ds(s * wb, wb)])

    return deg


def _prep_body(x_ref, dp_ref, dinv_ref, d2_ref, g_ref):
    deg = dp_ref[0, :, 0:1] + dp_ref[1, :, 0:1]
    dinv = jnp.where(deg > 0.0,
                     lax.rsqrt(jnp.maximum(deg, 1e-12)),
                     jnp.zeros_like(deg))
    dinv_ref[...] = dinv
    d2_ref[...] = jnp.broadcast_to(dinv * dinv, d2_ref.shape)
    xb = x_ref[...]
    z = jnp.zeros((xb.shape[0], RW - FW), jnp.float32)
    for g in range(NG):
        hg = xb[:, g * FW:(g + 1) * FW]
        g_ref[g] = jnp.concatenate([dinv * hg, z], axis=1)


def _final_body(kblk, x_ref, dinv_ref, accs_ref, w1_ref, b1_ref, gam_ref,
                bet_ref, mu_ref, var_ref, wm_ref, bm_ref, y_ref):
    xb = x_ref[...]
    dinv = dinv_ref[...]
    out = jnp.dot(xb, w1_ref[0], preferred_element_type=jnp.float32)
    t_m2 = None
    t_m1 = xb
    for k in range(1, kblk):
        a = jnp.concatenate(
            [accs_ref[k - 1, g][:, :FW] for g in range(NG)], axis=1)
        if k == 1:
            t = -dinv * a
        else:
            t = -2.0 * dinv * a - t_m2
        out = out + jnp.dot(t, w1_ref[k],
                            preferred_element_type=jnp.float32)
        t_m2, t_m1 = t_m1, t
    o = out + b1_ref[...]
    o = (o - mu_ref[...]) * lax.rsqrt(var_ref[...] + 1e-5) * gam_ref[...] \
        + bet_ref[...]
    h = jnp.maximum(o, 0.0)
    y_ref[...] = jnp.dot(h, wm_ref[...],
                         preferred_element_type=jnp.float32) + bm_ref[...]


def kernel(x, edge_index, W1, b1, gamma, beta, bn_mean, bn_var, Wmix, bmix):
    n, n_in = x.shape
    e = edge_index.shape[1]
    kblk = W1.shape[0]
    n_hid = W1.shape[2]
    n_out = Wmix.shape[2]
    nc, ns = _num_cores_subcores()

    per_tile = ns * SUPER
    nsup = -(-e // per_tile)
    nsup = -(-nsup // 6) * 6
    nsupd = nsup // nc
    epad = ns * nsup * SUPER
    pad = epad - e

    nacc = -(-(n + 1) // (ns * SUBW)) * (ns * SUBW)

    src = edge_index[0]
    dst = edge_index[1]
    src_g = jnp.concatenate([src, jnp.zeros((pad,), jnp.int32)])
    src_n = jnp.concatenate([src, jnp.full((pad,), n, jnp.int32)])
    dst_n = jnp.concatenate([dst, jnp.full((pad,), n, jnp.int32)])
    src4r = jnp.stack([src_g + (b * NG + g) * nacc
                       for b in range(2) for g in range(NG)]) \
               .reshape(2 * NG, epad // SUBW, SUBW)
    dstr = dst_n.reshape(epad // SUBW, SUBW)
    srcdr = src_n.reshape(epad // SUBW, SUBW)

    zero64 = jnp.zeros((64, RW), jnp.float32)
    ones128 = jnp.ones((SUBW, RW), jnp.float32)

    degp = _make_deg(n, nacc, nsupd, nc, ns)(srcdr, ones128, zero64)

    bsz = 2048
    assert nacc % bsz == 0
    grid = (nacc // bsz,)
    f32 = jnp.float32
    xp = jnp.concatenate([x, jnp.zeros((nacc - n, n_in), f32)])

    spec_x = pl.BlockSpec((bsz, n_in), lambda b: (b, 0))
    spec_deg = pl.BlockSpec((nc, bsz, RW), lambda b: (0, b, 0))
    spec_dinv = pl.BlockSpec((bsz, 1), lambda b: (b, 0))
    spec_d2 = pl.BlockSpec((bsz, RW), lambda b: (b, 0))
    spec_pack = pl.BlockSpec((NG, bsz, RW), lambda b: (0, b, 0))

    dinv, d2r, g0t = pl.pallas_call(
        _prep_body,
        grid=grid,
        in_specs=[spec_x, spec_deg],
        out_specs=[spec_dinv, spec_d2, spec_pack],
        out_shape=[
            jax.ShapeDtypeStruct((nacc, 1), f32),
            jax.ShapeDtypeStruct((nacc, RW), f32),
            jax.ShapeDtypeStruct((NG, nacc, RW), f32),
        ],
    )(xp, degp)

    accs, _gbuf = _make_mega(n, nacc, nsup, nc, ns, kblk)(
        g0t, src4r, dstr, d2r, zero64)

    spec_accs = pl.BlockSpec((kblk - 1, NG, bsz, RW),
                             lambda b: (0, 0, b, 0))
    spec_w1 = pl.BlockSpec((kblk, n_in, n_hid), lambda b: (0, 0, 0))
    spec_vec = pl.BlockSpec((1, n_hid), lambda b: (0, 0))
    spec_wm = pl.BlockSpec((n_hid, n_out), lambda b: (0, 0))
    spec_bm = pl.BlockSpec((1, n_out), lambda b: (0, 0))
    spec_y = pl.BlockSpec((bsz, n_out), lambda b: (b, 0))

    y = pl.pallas_call(
        functools.partial(_final_body, kblk),
        grid=grid,
        in_specs=[spec_x, spec_dinv, spec_accs, spec_w1, spec_vec,
                  spec_vec, spec_vec, spec_vec, spec_vec, spec_wm,
                  spec_bm],
        out_specs=spec_y,
        out_shape=jax.ShapeDtypeStruct((nacc, n_out), f32),
    )(xp, dinv, accs, W1, b1.reshape(1, n_hid), gamma.reshape(1, n_hid),
      beta.reshape(1, n_hid), bn_mean.reshape(1, n_hid),
      bn_var.reshape(1, n_hid), Wmix[0], bmix.reshape(1, n_out))
    return y[:n]


# speedup vs baseline: 1.9351x; 1.0190x over previous
import functools

import jax
import jax.numpy as jnp
from jax import lax
from jax.experimental import pallas as pl
from jax.experimental.pallas import tpu as pltpu
from jax.experimental.pallas import tpu_sc as plsc

RW = 16
FW = 12
NG = 2
SUBW = 128
SUB = 4
SUPER = SUB * SUBW


def _sc_mesh():
    return plsc.VectorSubcoreMesh(core_axis_name="c", subcore_axis_name="s")


def _num_cores_subcores():
    try:
        info = plsc.get_sparse_core_info()
        return info.num_cores, info.num_subcores
    except Exception:
        return 2, 16


def _make_mega(n, nacc, nsup, nc, ns, kblk):
    zblks = nacc // (ns * 64)
    wb = nacc // ns
    ublks = wb // SUBW

    @functools.partial(
        pl.kernel,
        out_type=[
            jax.ShapeDtypeStruct((kblk - 1, NG, nacc, RW), jnp.float32),
            jax.ShapeDtypeStruct((2 * NG * nacc, RW), jnp.float32),
        ],
        mesh=_sc_mesh(),
        compiler_params=pltpu.CompilerParams(
            use_tc_tiling_on_sc=False,
            internal_scratch_in_bytes=64 * 1024),
        scratch_types=(
            [pltpu.VMEM((SUB, SUBW), jnp.int32) for _ in range(6)]
            + [pltpu.VMEM((SUB, SUBW, RW), jnp.float32) for _ in range(2)]
            + [pltpu.VMEM((64, RW), jnp.float32)]
            + [pltpu.VMEM((SUBW, RW), jnp.float32) for _ in range(3)]
            + [pltpu.VMEM_SHARED((nacc, RW), jnp.float32)]
            + [pltpu.SemaphoreType.DMA for _ in range(7)]
        ),
    )
    def mega(g0t_hbm, srcr_hbm, dstr_hbm, d2r_hbm, zero_hbm,
             accs_hbm, gbuf_hbm, *scr):
        srcb = scr[0:3]
        dstb = scr[3:6]
        rows = scr[6:8]
        zero_v = scr[8]
        ub_a = scr[9]
        ub_b = scr[10]
        ub_d = scr[11]
        acc_sh = scr[12]
        isem = scr[13:16]
        gsem = scr[16:18]
        ssem = scr[18:20]
        c = lax.axis_index("c")
        s = lax.axis_index("s")
        gq = c
        pltpu.sync_copy(zero_hbm, zero_v)
        base = s * nsup * SUB
        r0w = s * wb

        def cbody(blk, carry):
            r0 = r0w + blk * SUBW
            pltpu.sync_copy(g0t_hbm.at[gq, pl.ds(r0, SUBW)], ub_a)
            pltpu.sync_copy(ub_a, gbuf_hbm.at[pl.ds(gq * nacc + r0, SUBW)])
            return carry

        lax.fori_loop(0, ublks, cbody, 0)
        plsc.subcore_barrier()

        def fire_idx(srcsel, i, b):
            r0 = base + i * SUB
            pltpu.async_copy(srcr_hbm.at[srcsel, pl.ds(r0, SUB)], srcb[b],
                             isem[b])
            pltpu.async_copy(dstr_hbm.at[pl.ds(r0, SUB)], dstb[b], isem[b])

        def drain_idx(srcsel, i, b):
            r0 = base + i * SUB
            pltpu.make_async_copy(srcr_hbm.at[srcsel, pl.ds(r0, SUB)],
                                  srcb[b], isem[b]).wait()
            pltpu.make_async_copy(dstr_hbm.at[pl.ds(r0, SUB)], dstb[b],
                                  isem[b]).wait()

        def fire_gather(bi, br):
            for j in range(SUB):
                pltpu.async_copy(gbuf_hbm.at[srcb[bi].at[j]],
                                 rows[br].at[j], gsem[br])

        def drain_gather(bi, br):
            for j in range(SUB):
                pltpu.make_async_copy(gbuf_hbm.at[srcb[bi].at[j]],
                                      rows[br].at[j], gsem[br]).wait()

        def fire_scat(bi, br):
            for j in range(SUB):
                pltpu.async_copy(rows[br].at[j], acc_sh.at[dstb[bi].at[j]],
                                 ssem[br], add=True)

        def drain_scat(bi, br):
            for j in range(SUB):
                pltpu.make_async_copy(rows[br].at[j],
                                      acc_sh.at[dstb[bi].at[j]],
                                      ssem[br]).wait()

        def kbody(k, carry):
            cur = lax.rem(k - 1, 2)
            nxt = lax.rem(k, 2)
            srcsel = cur * NG + gq
            scl = jnp.where(k == 1, jnp.float32(1.0), jnp.float32(2.0))

            def zbody(r, carry2):
                pltpu.sync_copy(
                    zero_v, acc_sh.at[pl.ds((s * zblks + r) * 64, 64)])
                return carry2

            lax.fori_loop(0, zblks, zbody, 0)
            plsc.subcore_barrier()

            fire_idx(srcsel, 0, 0)
            fire_idx(srcsel, 1, 1)
            drain_idx(srcsel, 0, 0)
            fire_gather(0, 0)

            def body(p, carry2):
                i0 = p * 6
                for q in range(6):
                    i = i0 + q
                    br = q % 2
                    brn = (q + 1) % 2
                    bi = q % 3
                    bi1 = (q + 1) % 3
                    bi2 = (q + 2) % 3
                    drain_gather(bi, br)
                    fire_scat(bi, br)
                    if q == 0:
                        @pl.when(p > 0)
                        def _():
                            drain_scat(bi2, brn)
                    else:
                        drain_scat(bi2, brn)

                    @pl.when(i + 1 < nsup)
                    def _():
                        drain_idx(srcsel, i + 1, bi1)
                        fire_gather(bi1, brn)

                    @pl.when(i + 2 < nsup)
                    def _():
                        fire_idx(srcsel, i + 2, bi2)
                return carry2

            lax.fori_loop(0, nsup // 6, body, 0)
            drain_scat((nsup - 1) % 3, (nsup - 1) % 2)
            plsc.subcore_barrier()

            def ubody(blk, carry2):
                r0 = r0w + blk * SUBW
                pltpu.sync_copy(acc_sh.at[pl.ds(r0, SUBW)], ub_a)
                pltpu.sync_copy(
                    gbuf_hbm.at[pl.ds((nxt * NG + gq) * nacc + r0, SUBW)],
                    ub_b)
                pltpu.sync_copy(d2r_hbm.at[pl.ds(r0, SUBW)], ub_d)

                def vbody(v, carry3):
                    a = ub_a[v]
                    gp = ub_b[v]
                    d2 = ub_d[v]
                    gn = (-scl) * d2 * a - \
                        jnp.where(k > 1, gp, jnp.zeros_like(gp))
                    ub_b[v] = gn
                    return carry3

                lax.fori_loop(0, SUBW, vbody, 0)
                pltpu.sync_copy(
                    ub_b,
                    gbuf_hbm.at[pl.ds((nxt * NG + gq) * nacc + r0, SUBW)])
                pltpu.sync_copy(ub_a,
                                accs_hbm.at[k - 1, gq, pl.ds(r0, SUBW)])
                return carry2

            lax.fori_loop(0, ublks, ubody, 0)
            plsc.subcore_barrier()
            return carry

        lax.fori_loop(1, kblk, kbody, 0)

    return mega


def _make_deg(n, nacc, nsupd, nc, ns):
    zblks = nacc // (ns * 64)
    wb = nacc // ns

    @functools.partial(
        pl.kernel,
        out_type=jax.ShapeDtypeStruct((nc, nacc, RW), jnp.float32),
        mesh=_sc_mesh(),
        compiler_params=pltpu.CompilerParams(
            use_tc_tiling_on_sc=False,
            internal_scratch_in_bytes=64 * 1024),
        scratch_types=(
            [pltpu.VMEM((SUB, SUBW), jnp.int32) for _ in range(3)]
            + [pltpu.VMEM((SUBW, RW), jnp.float32),
               pltpu.VMEM((64, RW), jnp.float32),
               pltpu.VMEM_SHARED((nacc, RW), jnp.float32)]
            + [pltpu.SemaphoreType.DMA for _ in range(6)]
        ),
    )
    def deg(srcr_hbm, ones_hbm, zero_hbm, out_hbm, *scr):
        idxb = scr[0:3]
        ones_v = scr[3]
        zero_v = scr[4]
        acc_sh = scr[5]
        isem = scr[6:9]
        ssem = scr[9:12]
        c = lax.axis_index("c")
        s = lax.axis_index("s")

        pltpu.sync_copy(zero_hbm, zero_v)
        pltpu.sync_copy(ones_hbm, ones_v)
        base = ((c * ns + s) * nsupd) * SUB

        def fire_idx(i, b):
            pltpu.async_copy(srcr_hbm.at[pl.ds(base + i * SUB, SUB)],
                             idxb[b], isem[b])

        def drain_idx(i, b):
            pltpu.make_async_copy(srcr_hbm.at[pl.ds(base + i * SUB, SUB)],
                                  idxb[b], isem[b]).wait()

        def fire_scat(b):
            for j in range(SUB):
                pltpu.async_copy(ones_v, acc_sh.at[idxb[b].at[j]],
                                 ssem[b], add=True)

        def drain_scat(b):
            for j in range(SUB):
                pltpu.make_async_copy(ones_v, acc_sh.at[idxb[b].at[j]],
                                      ssem[b]).wait()

        def zbody(r, carry):
            pltpu.sync_copy(zero_v,
                            acc_sh.at[pl.ds((s * zblks + r) * 64, 64)])
            return carry

        lax.fori_loop(0, zblks, zbody, 0)
        plsc.subcore_barrier()

        fire_idx(0, 0)
        fire_idx(1, 1)

        def body(p, carry):
            i0 = p * 3
            for q in range(3):
                i = i0 + q
                b = q
                b1 = (q + 1) % 3
                b2 = (q + 2) % 3
                if q == 0:
                    @pl.when(p > 0)
                    def _():
                        drain_scat(b2)
                else:
                    drain_scat(b2)

                @pl.when(i + 2 < nsupd)
                def _():
                    fire_idx(i + 2, b2)

                drain_idx(i, b)
                fire_scat(b)
            return carry

        lax.fori_loop(0, nsupd // 3, body, 0)
        drain_scat(2)
        plsc.subcore_barrier()
        pltpu.sync_copy(acc_sh.at[pl.ds(s * wb, wb)],
                        out_hbm.at[c, pl.ds(s * wb, wb)])

    return deg


def _prep_body(x_ref, dp_ref, dinv_ref, d2_ref, g_ref):
    deg = dp_ref[0, :, 0:1] + dp_ref[1, :, 0:1]
    dinv = jnp.where(deg > 0.0,
                     lax.rsqrt(jnp.maximum(deg, 1e-12)),
                     jnp.zeros_like(deg))
    dinv_ref[...] = dinv
    d2_ref[...] = jnp.broadcast_to(dinv * dinv, d2_ref.shape)
    xb = x_ref[...]
    z = jnp.zeros((xb.shape[0], RW - FW), jnp.float32)
    for g in range(NG):
        hg = xb[:, g * FW:(g + 1) * FW]
        g_ref[g] = jnp.concatenate([dinv * hg, z], axis=1)


def _final_body(kblk, x_ref, dinv_ref, accs_ref, w1_ref, b1_ref, gam_ref,
                bet_ref, mu_ref, var_ref, wm_ref, bm_ref, y_ref):
    xb = x_ref[...]
    dinv = dinv_ref[...]
    out = jnp.dot(xb, w1_ref[0], preferred_element_type=jnp.float32)
    t_m2 = None
    t_m1 = xb
    for k in range(1, kblk):
        a = jnp.concatenate(
            [accs_ref[k - 1, g][:, :FW] for g in range(NG)], axis=1)
        if k == 1:
            t = -dinv * a
        else:
            t = -2.0 * dinv * a - t_m2
        out = out + jnp.dot(t, w1_ref[k],
                            preferred_element_type=jnp.float32)
        t_m2, t_m1 = t_m1, t
    o = out + b1_ref[...]
    o = (o - mu_ref[...]) * lax.rsqrt(var_ref[...] + 1e-5) * gam_ref[...] \
        + bet_ref[...]
    h = jnp.maximum(o, 0.0)
    y_ref[...] = jnp.dot(h, wm_ref[...],
                         preferred_element_type=jnp.float32) + bm_ref[...]


def kernel(x, edge_index, W1, b1, gamma, beta, bn_mean, bn_var, Wmix, bmix):
    n, n_in = x.shape
    e = edge_index.shape[1]
    kblk = W1.shape[0]
    n_hid = W1.shape[2]
    n_out = Wmix.shape[2]
    nc, ns = _num_cores_subcores()

    per_tile = ns * SUPER
    nsup = -(-e // per_tile)
    nsup = -(-nsup // 6) * 6
    nsupd = nsup // nc
    epad = ns * nsup * SUPER
    pad = epad - e

    nacc = -(-(n + 1) // (ns * SUBW)) * (ns * SUBW)

    src = edge_index[0]
    dst = edge_index[1]
    src_g = jnp.concatenate([src, jnp.zeros((pad,), jnp.int32)])
    src_n = jnp.concatenate([src, jnp.full((pad,), n, jnp.int32)])
    dst_n = jnp.concatenate([dst, jnp.full((pad,), n, jnp.int32)])
    src4r = jnp.stack([src_g + (b * NG + g) * nacc
                       for b in range(2) for g in range(NG)]) \
               .reshape(2 * NG, epad // SUBW, SUBW)
    dstr = dst_n.reshape(epad // SUBW, SUBW)
    srcdr = src_n.reshape(epad // SUBW, SUBW)

    zero64 = jnp.zeros((64, RW), jnp.float32)
    ones128 = jnp.ones((SUBW, RW), jnp.float32)

    degp = _make_deg(n, nacc, nsupd, nc, ns)(srcdr, ones128, zero64)

    bsz = 2048
    assert nacc % bsz == 0
    grid = (nacc // bsz,)
    f32 = jnp.float32
    xp = jnp.concatenate([x, jnp.zeros((nacc - n, n_in), f32)])

    spec_x = pl.BlockSpec((bsz, n_in), lambda b: (b, 0))
    spec_deg = pl.BlockSpec((nc, bsz, RW), lambda b: (0, b, 0))
    spec_dinv = pl.BlockSpec((bsz, 1), lambda b: (b, 0))
    spec_d2 = pl.BlockSpec((bsz, RW), lambda b: (b, 0))
    spec_pack = pl.BlockSpec((NG, bsz, RW), lambda b: (0, b, 0))

    dinv, d2r, g0t = pl.pallas_call(
        _prep_body,
        grid=grid,
        in_specs=[spec_x, spec_deg],
        out_specs=[spec_dinv, spec_d2, spec_pack],
        out_shape=[
            jax.ShapeDtypeStruct((nacc, 1), f32),
            jax.ShapeDtypeStruct((nacc, RW), f32),
            jax.ShapeDtypeStruct((NG, nacc, RW), f32),
        ],
    )(xp, degp)

    accs, _gbuf = _make_mega(n, nacc, nsup, nc, ns, kblk)(
        g0t, src4r, dstr, d2r, zero64)

    spec_accs = pl.BlockSpec((kblk - 1, NG, bsz, RW),
                             lambda b: (0, 0, b, 0))
    spec_w1 = pl.BlockSpec((kblk, n_in, n_hid), lambda b: (0, 0, 0))
    spec_vec = pl.BlockSpec((1, n_hid), lambda b: (0, 0))
    spec_wm = pl.BlockSpec((n_hid, n_out), lambda b: (0, 0))
    spec_bm = pl.BlockSpec((1, n_out), lambda b: (0, 0))
    spec_y = pl.BlockSpec((bsz, n_out), lambda b: (b, 0))

    y = pl.pallas_call(
        functools.partial(_final_body, kblk),
        grid=grid,
        in_specs=[spec_x, spec_dinv, spec_accs, spec_w1, spec_vec,
                  spec_vec, spec_vec, spec_vec, spec_vec, spec_wm,
                  spec_bm],
        out_specs=spec_y,
        out_shape=jax.ShapeDtypeStruct((nacc, n_out), f32),
    )(xp, dinv, accs, W1, b1.reshape(1, n_hid), gamma.reshape(1, n_hid),
      beta.reshape(1, n_hid), bn_mean.reshape(1, n_hid),
      bn_var.reshape(1, n_hid), Wmix[0], bmix.reshape(1, n_out))
    return y[:n]
